# Initial kernel scaffold; baseline (speedup 1.0000x reference)
import jax
import jax.numpy as jnp
from jax.experimental import pallas as pl


def kernel(x, table, W, b):
    raise NotImplementedError("write your pallas kernel here")


# speedup vs baseline: 2.5304x; 2.5304x over previous
import functools

import jax
import jax.numpy as jnp
from jax import lax
from jax.experimental import pallas as pl
from jax.experimental.pallas import tpu as pltpu
from jax.experimental.pallas import tpu_sc as plsc

NUM_EMBEDDINGS = 1000000
EMBED_DIM = 64
DENSE_OUT = 64
BATCH = 16384
HIST = 50

NC = 2
NS = 16
NW = NC * NS

BAGS_PER_TILE = BATCH // NW
CHUNK_BAGS = 16
CHUNKS_PER_TILE = BAGS_PER_TILE // CHUNK_BAGS
IDX_PER_CHUNK = CHUNK_BAGS * HIST
GATHER_W = 80
GATHERS_PER_CHUNK = IDX_PER_CHUNK // GATHER_W
IDX_ROWS_PER_TILE = CHUNKS_PER_TILE * GATHERS_PER_CHUNK


def _sc_pool(x2d, table):
    mesh = plsc.VectorSubcoreMesh(core_axis_name="c", subcore_axis_name="s")

    @functools.partial(
        pl.kernel,
        mesh=mesh,
        compiler_params=pltpu.CompilerParams(use_tc_tiling_on_sc=False),
        out_type=jax.ShapeDtypeStruct((BATCH, EMBED_DIM), jnp.float32),
        scratch_types=[
            pltpu.VMEM((IDX_ROWS_PER_TILE, GATHER_W), jnp.int32),
            pltpu.VMEM((IDX_PER_CHUNK, EMBED_DIM), jnp.float32),
            pltpu.VMEM((CHUNK_BAGS, EMBED_DIM), jnp.float32),
            pltpu.SemaphoreType.DMA,
        ],
    )
    def sc_pool(x_hbm, table_hbm, out_hbm, idx_v, rows_v, acc_v, sem):
        wid = lax.axis_index("s") * NC + lax.axis_index("c")
        idx_row0 = wid * IDX_ROWS_PER_TILE
        bag0 = wid * BAGS_PER_TILE

        pltpu.sync_copy(x_hbm.at[pl.ds(idx_row0, IDX_ROWS_PER_TILE)], idx_v)

        def chunk_body(c, carry):
            handles = [
                pltpu.async_copy(
                    table_hbm.at[idx_v.at[c * GATHERS_PER_CHUNK + k]],
                    rows_v.at[pl.ds(k * GATHER_W, GATHER_W)],
                    sem)
                for k in range(GATHERS_PER_CHUNK)
            ]
            for h in handles:
                h.wait()

            def bag_body(j, carry2):
                rbase = j * HIST

                def r_body(ri, accs):
                    out = list(accs)
                    for u in range(10):
                        row = rbase + ri * 10 + u
                        for dk in range(4):
                            out[dk] = out[dk] + rows_v[row, pl.ds(dk * 16, 16)]
                    return tuple(out)

                z = jnp.zeros((16,), jnp.float32)
                accs = lax.fori_loop(0, HIST // 10, r_body, (z, z, z, z))
                for dk in range(4):
                    acc_v[j, pl.ds(dk * 16, 16)] = accs[dk]
                return carry2

            lax.fori_loop(0, CHUNK_BAGS, bag_body, 0)
            pltpu.sync_copy(acc_v,
                            out_hbm.at[pl.ds(bag0 + c * CHUNK_BAGS,
                                             CHUNK_BAGS)])
            return carry

        lax.fori_loop(0, CHUNKS_PER_TILE, chunk_body, 0)

    return sc_pool(x2d, table)


def _tc_body(p_ref, w_ref, b_ref, o_ref):
    p = p_ref[:] * (1.0 / HIST)
    logits = lax.dot_general(p, w_ref[:], (((1,), (1,)), ((), ())),
                             preferred_element_type=jnp.float32)
    logits = logits + b_ref[:]
    m = jnp.max(logits, axis=1, keepdims=True)
    e = jnp.exp(logits - m)
    o_ref[:] = e / jnp.sum(e, axis=1, keepdims=True)


_TC_BLOCK = 1024


def _tc_dense(pooled, W, b2):
    return pl.pallas_call(
        _tc_body,
        grid=(BATCH // _TC_BLOCK,),
        in_specs=[
            pl.BlockSpec((_TC_BLOCK, EMBED_DIM), lambda i: (i, 0)),
            pl.BlockSpec((DENSE_OUT, EMBED_DIM), lambda i: (0, 0)),
            pl.BlockSpec((1, DENSE_OUT), lambda i: (0, 0)),
        ],
        out_specs=pl.BlockSpec((_TC_BLOCK, DENSE_OUT), lambda i: (i, 0)),
        out_shape=jax.ShapeDtypeStruct((BATCH, DENSE_OUT), jnp.float32),
    )(pooled, W, b2)


@jax.jit
def kernel(x, table, W, b):
    x2d = x.astype(jnp.int32).reshape(BATCH * HIST // GATHER_W, GATHER_W)
    pooled = _sc_pool(x2d, table)
    return _tc_dense(pooled, W, b.reshape(1, DENSE_OUT))


# speedup vs baseline: 2.7387x; 1.0823x over previous
import functools

import jax
import jax.numpy as jnp
from jax import lax
from jax.experimental import pallas as pl
from jax.experimental.pallas import tpu as pltpu
from jax.experimental.pallas import tpu_sc as plsc

NUM_EMBEDDINGS = 1000000
EMBED_DIM = 64
DENSE_OUT = 64
BATCH = 16384
HIST = 50

NC = 2
NS = 16
NW = NC * NS

BAGS_PER_TILE = BATCH // NW
CHUNK_BAGS = 8
CHUNKS_PER_TILE = BAGS_PER_TILE // CHUNK_BAGS
IDX_PER_CHUNK = CHUNK_BAGS * HIST
GATHER_W = 80
GATHERS_PER_CHUNK = IDX_PER_CHUNK // GATHER_W
IDX_ROWS_PER_TILE = BAGS_PER_TILE * HIST // GATHER_W


def _sc_pool(x2d, table):
    mesh = plsc.VectorSubcoreMesh(core_axis_name="c", subcore_axis_name="s")

    @functools.partial(
        pl.kernel,
        mesh=mesh,
        compiler_params=pltpu.CompilerParams(use_tc_tiling_on_sc=False),
        out_type=jax.ShapeDtypeStruct((BATCH, EMBED_DIM), jnp.float32),
        scratch_types=[
            pltpu.VMEM((IDX_ROWS_PER_TILE, GATHER_W), jnp.int32),
            pltpu.VMEM((2, IDX_PER_CHUNK, EMBED_DIM), jnp.float32),
            pltpu.VMEM((BAGS_PER_TILE, EMBED_DIM), jnp.float32),
            pltpu.SemaphoreType.DMA,
            pltpu.SemaphoreType.DMA,
        ],
    )
    def sc_pool(x_hbm, table_hbm, out_hbm, idx_v, rows_v, acc_v, sem0, sem1):
        wid = lax.axis_index("s") * NC + lax.axis_index("c")
        idx_row0 = wid * IDX_ROWS_PER_TILE
        bag0 = wid * BAGS_PER_TILE
        sems = (sem0, sem1)
        rows_b = (rows_v.at[0], rows_v.at[1])

        pltpu.sync_copy(x_hbm.at[pl.ds(idx_row0, IDX_ROWS_PER_TILE)], idx_v)

        def fire(chunk, b):
            for k in range(GATHERS_PER_CHUNK):
                pltpu.async_copy(
                    table_hbm.at[idx_v.at[chunk * GATHERS_PER_CHUNK + k]],
                    rows_b[b].at[pl.ds(k * GATHER_W, GATHER_W)],
                    sems[b])

        def drain(chunk, b):
            for k in range(GATHERS_PER_CHUNK):
                pltpu.make_async_copy(
                    table_hbm.at[idx_v.at[chunk * GATHERS_PER_CHUNK + k]],
                    rows_b[b].at[pl.ds(k * GATHER_W, GATHER_W)],
                    sems[b]).wait()

        def compute(chunk, b):
            rb = rows_b[b]

            def bag_body(j, carry):
                rbase = j * HIST

                def r_body(ri, accs):
                    out = list(accs)
                    for u in range(10):
                        row = rbase + ri * 10 + u
                        for dk in range(4):
                            out[dk] = out[dk] + rb[row, pl.ds(dk * 16, 16)]
                    return tuple(out)

                z = jnp.zeros((16,), jnp.float32)
                accs = lax.fori_loop(0, HIST // 10, r_body, (z, z, z, z))
                gbag = chunk * CHUNK_BAGS + j
                for dk in range(4):
                    acc_v[gbag, pl.ds(dk * 16, 16)] = accs[dk]
                return carry

            lax.fori_loop(0, CHUNK_BAGS, bag_body, 0)

        fire(0, 0)
        fire(1, 1)

        def step(c, carry):
            for b in range(2):
                chunk = 2 * c + b
                drain(chunk, b)
                compute(chunk, b)

                @pl.when(chunk < CHUNKS_PER_TILE - 2)
                def _():
                    fire(chunk + 2, b)
            return carry

        lax.fori_loop(0, CHUNKS_PER_TILE // 2, step, 0)
        pltpu.sync_copy(acc_v, out_hbm.at[pl.ds(bag0, BAGS_PER_TILE)])

    return sc_pool(x2d, table)


def _tc_body(p_ref, w_ref, b_ref, o_ref):
    p = p_ref[:] * (1.0 / HIST)
    logits = lax.dot_general(p, w_ref[:], (((1,), (1,)), ((), ())),
                             preferred_element_type=jnp.float32)
    logits = logits + b_ref[:]
    m = jnp.max(logits, axis=1, keepdims=True)
    e = jnp.exp(logits - m)
    o_ref[:] = e / jnp.sum(e, axis=1, keepdims=True)


_TC_BLOCK = 1024


def _tc_dense(pooled, W, b2):
    return pl.pallas_call(
        _tc_body,
        grid=(BATCH // _TC_BLOCK,),
        in_specs=[
            pl.BlockSpec((_TC_BLOCK, EMBED_DIM), lambda i: (i, 0)),
            pl.BlockSpec((DENSE_OUT, EMBED_DIM), lambda i: (0, 0)),
            pl.BlockSpec((1, DENSE_OUT), lambda i: (0, 0)),
        ],
        out_specs=pl.BlockSpec((_TC_BLOCK, DENSE_OUT), lambda i: (i, 0)),
        out_shape=jax.ShapeDtypeStruct((BATCH, DENSE_OUT), jnp.float32),
    )(pooled, W, b2)


@jax.jit
def kernel(x, table, W, b):
    x2d = x.astype(jnp.int32).reshape(BATCH * HIST // GATHER_W, GATHER_W)
    pooled = _sc_pool(x2d, table)
    return _tc_dense(pooled, W, b.reshape(1, DENSE_OUT))
